# baseline (speedup 1.0000x reference)
import functools
import math

import jax
import jax.numpy as jnp
from jax import lax
from jax.experimental import pallas as pl
from jax.experimental.pallas import tpu as pltpu

B, L, T, F = 8, 25, 512, 1024
H = 8
DH = F // H
TOPK = 8
BL = B * L


def _mean_body(x_ref, o_ref):
    o_ref[...] = jnp.sum(x_ref[0], axis=0)[None, None, :] * (1.0 / T)


def _mean_over_t(x2):
    return pl.pallas_call(
        _mean_body,
        grid=(BL,),
        in_specs=[pl.BlockSpec((1, T, F), lambda i: (i, 0, 0))],
        out_specs=pl.BlockSpec((1, 1, F), lambda i: (i, 0, 0)),
        out_shape=jax.ShapeDtypeStruct((BL, 1, F), jnp.float32),
    )(x2)


def _head_body(xm_ref, wi_ref, bi_ref, wo_ref, bo_ref, w1_ref, b1_ref,
               w2_ref, b2_ref, out_ref):
    xm = xm_ref[...]
    f32 = jnp.float32

    def dot_t(a, w):
        return lax.dot_general(a, w, (((1,), (1,)), ((), ())),
                               preferred_element_type=f32)

    qkv = dot_t(xm, wi_ref[...]) + bi_ref[...]
    scale = 1.0 / math.sqrt(DH)

    rows = []
    for b in range(B):
        r0 = b * L
        head_outs = []
        for h in range(H):
            c0 = h * DH
            q = qkv[r0:r0 + L, c0:c0 + DH]
            k = qkv[r0:r0 + L, F + c0:F + c0 + DH]
            v = qkv[r0:r0 + L, 2 * F + c0:2 * F + c0 + DH]
            s = dot_t(q, k) * scale
            m = jnp.max(s, axis=1, keepdims=True)
            e = jnp.exp(s - m)
            a = e / jnp.sum(e, axis=1, keepdims=True)
            oh = lax.dot_general(a, v, (((1,), (0,)), ((), ())),
                                 preferred_element_type=f32)
            head_outs.append(oh)
        rows.append(jnp.concatenate(head_outs, axis=1))
    o = jnp.concatenate(rows, axis=0)

    att = dot_t(o, wo_ref[...]) + bo_ref[...]
    g = jnp.concatenate(
        [jnp.mean(att[b * L:(b + 1) * L, :], axis=0, keepdims=True)
         for b in range(B)], axis=0)

    hmid = jnp.maximum(dot_t(g, w1_ref[...]) + b1_ref[...], 0.0)
    logits = dot_t(hmid, w2_ref[...]) + b2_ref[...]

    lm = jnp.max(logits, axis=1, keepdims=True)
    ex = jnp.exp(logits - lm)
    probs = ex / jnp.sum(ex, axis=1, keepdims=True)

    iot = lax.broadcasted_iota(jnp.int32, (B, L), 1)
    work = probs
    mask = jnp.zeros((B, L), dtype=jnp.bool_)
    for _ in range(TOPK):
        cur = jnp.max(work, axis=1, keepdims=True)
        cand = jnp.where(work == cur, iot, jnp.int32(2 ** 30))
        sel = jnp.min(cand, axis=1, keepdims=True)
        hit = iot == sel
        mask = mask | hit
        work = jnp.where(hit, -1.0, work)

    kept = jnp.where(mask, probs, 0.0)
    denom = jnp.sum(kept, axis=1, keepdims=True)
    final = kept / denom

    outs = []
    for b in range(B):
        w_b = final[b:b + 1, :]
        outs.append(lax.dot_general(w_b, xm[b * L:(b + 1) * L, :],
                                    (((1,), (0,)), ((), ())),
                                    preferred_element_type=f32))
    out_ref[...] = jnp.concatenate(outs, axis=0)


def _gating_head(xm, wi, bi, wo, bo, w1, b1, w2, b2):
    return pl.pallas_call(
        _head_body,
        out_shape=jax.ShapeDtypeStruct((B, F), jnp.float32),
    )(xm, wi, bi, wo, bo, w1, b1, w2, b2)


@jax.jit
def kernel(x, in_proj_w, in_proj_b, out_proj_w, out_proj_b, W1, b1, W2, b2):
    x2 = x.reshape(BL, T, F)
    xm = _mean_over_t(x2).reshape(BL, F)
    return _gating_head(
        xm, in_proj_w, in_proj_b.reshape(1, -1), out_proj_w,
        out_proj_b.reshape(1, -1), W1, b1.reshape(1, -1), W2,
        b2.reshape(1, -1))


# speedup vs baseline: 1.2530x; 1.2530x over previous
import functools
import math

import jax
import jax.numpy as jnp
from jax import lax
from jax.experimental import pallas as pl
from jax.experimental.pallas import tpu as pltpu

B, L, T, F = 8, 25, 512, 1024
H = 8
DH = F // H
TOPK = 8
BL = B * L


ROWS = 8


def _mean_body(x_ref, o_ref):
    o_ref[...] = jnp.sum(x_ref[...], axis=1) * (1.0 / T)


def _mean_over_t(x2):
    return pl.pallas_call(
        _mean_body,
        grid=(BL // ROWS,),
        in_specs=[pl.BlockSpec((ROWS, T, F), lambda i: (i, 0, 0))],
        out_specs=pl.BlockSpec((ROWS, F), lambda i: (i, 0)),
        out_shape=jax.ShapeDtypeStruct((BL, F), jnp.float32),
    )(x2)


def _head_body(xm_ref, wi_ref, bi_ref, wo_ref, bo_ref, w1_ref, b1_ref,
               w2_ref, b2_ref, out_ref):
    xm = xm_ref[...]
    f32 = jnp.float32

    def dot_t(a, w):
        return lax.dot_general(a, w, (((1,), (1,)), ((), ())),
                               preferred_element_type=f32)

    qkv = dot_t(xm, wi_ref[...]) + bi_ref[...]
    scale = 1.0 / math.sqrt(DH)

    rows = []
    for b in range(B):
        r0 = b * L
        head_outs = []
        for h in range(H):
            c0 = h * DH
            q = qkv[r0:r0 + L, c0:c0 + DH]
            k = qkv[r0:r0 + L, F + c0:F + c0 + DH]
            v = qkv[r0:r0 + L, 2 * F + c0:2 * F + c0 + DH]
            s = dot_t(q, k) * scale
            m = jnp.max(s, axis=1, keepdims=True)
            e = jnp.exp(s - m)
            a = e / jnp.sum(e, axis=1, keepdims=True)
            oh = lax.dot_general(a, v, (((1,), (0,)), ((), ())),
                                 preferred_element_type=f32)
            head_outs.append(oh)
        rows.append(jnp.concatenate(head_outs, axis=1))
    o = jnp.concatenate(rows, axis=0)

    att = dot_t(o, wo_ref[...]) + bo_ref[...]
    g = jnp.concatenate(
        [jnp.mean(att[b * L:(b + 1) * L, :], axis=0, keepdims=True)
         for b in range(B)], axis=0)

    hmid = jnp.maximum(dot_t(g, w1_ref[...]) + b1_ref[...], 0.0)
    logits = dot_t(hmid, w2_ref[...]) + b2_ref[...]

    lm = jnp.max(logits, axis=1, keepdims=True)
    ex = jnp.exp(logits - lm)
    probs = ex / jnp.sum(ex, axis=1, keepdims=True)

    iot = lax.broadcasted_iota(jnp.int32, (B, L), 1)
    work = probs
    mask = jnp.zeros((B, L), dtype=jnp.bool_)
    for _ in range(TOPK):
        cur = jnp.max(work, axis=1, keepdims=True)
        cand = jnp.where(work == cur, iot, jnp.int32(2 ** 30))
        sel = jnp.min(cand, axis=1, keepdims=True)
        hit = iot == sel
        mask = mask | hit
        work = jnp.where(hit, -1.0, work)

    kept = jnp.where(mask, probs, 0.0)
    denom = jnp.sum(kept, axis=1, keepdims=True)
    final = kept / denom

    outs = []
    for b in range(B):
        w_b = final[b:b + 1, :]
        outs.append(lax.dot_general(w_b, xm[b * L:(b + 1) * L, :],
                                    (((1,), (0,)), ((), ())),
                                    preferred_element_type=f32))
    out_ref[...] = jnp.concatenate(outs, axis=0)


def _gating_head(xm, wi, bi, wo, bo, w1, b1, w2, b2):
    return pl.pallas_call(
        _head_body,
        out_shape=jax.ShapeDtypeStruct((B, F), jnp.float32),
    )(xm, wi, bi, wo, bo, w1, b1, w2, b2)


@jax.jit
def kernel(x, in_proj_w, in_proj_b, out_proj_w, out_proj_b, W1, b1, W2, b2):
    x2 = x.reshape(BL, T, F)
    xm = _mean_over_t(x2)
    return _gating_head(
        xm, in_proj_w, in_proj_b.reshape(1, -1), out_proj_w,
        out_proj_b.reshape(1, -1), W1, b1.reshape(1, -1), W2,
        b2.reshape(1, -1))


# speedup vs baseline: 1.3815x; 1.1025x over previous
import functools
import math

import jax
import jax.numpy as jnp
from jax import lax
from jax.experimental import pallas as pl
from jax.experimental.pallas import tpu as pltpu

B, L, T, F = 8, 25, 512, 1024
H = 8
DH = F // H
TOPK = 8
BL = B * L
ROWS = 8
NSTEP = BL // ROWS
NEG = -1e30


def _dot_t(a, w):
    return lax.dot_general(a, w, (((1,), (1,)), ((), ())),
                           preferred_element_type=jnp.float32)


def _dot(a, b):
    return lax.dot_general(a, b, (((1,), (0,)), ((), ())),
                           preferred_element_type=jnp.float32)


def _body(x_ref, wi_ref, bi_ref, wo_ref, bo_ref, w1_ref, b1_ref,
          w2_ref, b2_ref, out_ref, xm_s, qkv_s):
    i = pl.program_id(0)
    rows = jnp.sum(x_ref[...], axis=1) * (1.0 / T)
    xm_s[pl.ds(i * ROWS, ROWS), :] = rows
    qkv_s[pl.ds(i * ROWS, ROWS), :] = _dot_t(rows, wi_ref[...]) + bi_ref[...]

    @pl.when(i == NSTEP - 1)
    def _tail():
        qkv = qkv_s[...]
        xm = xm_s[...]
        scale = 1.0 / math.sqrt(DH)

        r_id = lax.broadcasted_iota(jnp.int32, (BL, BL), 0) // L
        c_id = lax.broadcasted_iota(jnp.int32, (BL, BL), 1) // L
        same_b = r_id == c_id
        head_outs = []
        for h in range(H):
            c0 = h * DH
            q = qkv[:, c0:c0 + DH]
            k = qkv[:, F + c0:F + c0 + DH]
            v = qkv[:, 2 * F + c0:2 * F + c0 + DH]
            s = jnp.where(same_b, _dot_t(q, k) * scale, NEG)
            m = jnp.max(s, axis=1, keepdims=True)
            e = jnp.exp(s - m)
            a = e / jnp.sum(e, axis=1, keepdims=True)
            head_outs.append(_dot(a, v))
        o = jnp.concatenate(head_outs, axis=1)

        att = _dot_t(o, wo_ref[...]) + bo_ref[...]

        br = lax.broadcasted_iota(jnp.int32, (B, BL), 0)
        bc = lax.broadcasted_iota(jnp.int32, (B, BL), 1) // L
        pool = jnp.where(br == bc, jnp.float32(1.0 / L), 0.0)
        g = _dot(pool, att)

        hmid = jnp.maximum(_dot_t(g, w1_ref[...]) + b1_ref[...], 0.0)
        logits = _dot_t(hmid, w2_ref[...]) + b2_ref[...]

        lm = jnp.max(logits, axis=1, keepdims=True)
        ex = jnp.exp(logits - lm)
        probs = ex / jnp.sum(ex, axis=1, keepdims=True)

        iot = lax.broadcasted_iota(jnp.int32, (B, L), 1)
        work = probs
        mask = jnp.zeros((B, L), dtype=jnp.bool_)
        for _ in range(TOPK):
            cur = jnp.max(work, axis=1, keepdims=True)
            cand = jnp.where(work == cur, iot, jnp.int32(2 ** 30))
            sel = jnp.min(cand, axis=1, keepdims=True)
            hit = iot == sel
            mask = mask | hit
            work = jnp.where(hit, -1.0, work)

        kept = jnp.where(mask, probs, 0.0)
        denom = jnp.sum(kept, axis=1, keepdims=True)
        final = kept / denom

        wfull = jnp.where(br == bc, jnp.concatenate([final] * B, axis=1), 0.0)
        out_ref[...] = _dot(wfull, xm)


@jax.jit
def kernel(x, in_proj_w, in_proj_b, out_proj_w, out_proj_b, W1, b1, W2, b2):
    x2 = x.reshape(BL, T, F)
    const = lambda i: (0, 0)
    return pl.pallas_call(
        _body,
        grid=(NSTEP,),
        in_specs=[
            pl.BlockSpec((ROWS, T, F), lambda i: (i, 0, 0)),
            pl.BlockSpec((3 * F, F), const),
            pl.BlockSpec((1, 3 * F), const),
            pl.BlockSpec((F, F), const),
            pl.BlockSpec((1, F), const),
            pl.BlockSpec((F, F), const),
            pl.BlockSpec((1, F), const),
            pl.BlockSpec((L, F), const),
            pl.BlockSpec((1, L), const),
        ],
        out_specs=pl.BlockSpec((B, F), const),
        out_shape=jax.ShapeDtypeStruct((B, F), jnp.float32),
        scratch_shapes=[
            pltpu.VMEM((BL, F), jnp.float32),
            pltpu.VMEM((BL, 3 * F), jnp.float32),
        ],
    )(x2, in_proj_w, in_proj_b.reshape(1, -1), out_proj_w,
      out_proj_b.reshape(1, -1), W1, b1.reshape(1, -1), W2,
      b2.reshape(1, -1))
